# baseline (speedup 1.0000x reference)
import functools

import jax
import jax.numpy as jnp
from jax import lax
from jax.experimental import pallas as pl
from jax.experimental.pallas import tpu as pltpu
from jax.experimental.pallas import tpu_sc as plsc

_INFO = plsc.get_sparse_core_info()
_NC, _NS = _INFO.num_cores, _INFO.num_subcores
_NW = _NC * _NS

_B, _S, _H = 4, 2048, 1024
_PPW = _S // _NW
_CP = 8
_NCHUNK = _PPW // _CP
_CR = _CP * _B
_HV = _H // 16
_NBUF = 3


def _body(ids_hbm, mask_hbm, tok_hbm, pos_hbm, out_hbm, mout_hbm,
          idx_v, tok0, tok1, tok2, pos0, pos1, pos2, m_v,
          sg0, sg1, sg2, sp0, sp1, sp2, so0, so1, so2, sm, si):
    wid = lax.axis_index("s") * _NC + lax.axis_index("c")
    p0 = wid * _PPW

    toks = (tok0, tok1, tok2)
    poss = (pos0, pos1, pos2)
    sgs = (sg0, sg1, sg2)
    sps = (sp0, sp1, sp2)
    sos = (so0, so1, so2)

    def stage_idx(cs, sem):
        return [pltpu.async_copy(
            ids_hbm.at[b].at[pl.ds(p0 + c * _CP, _CP)],
            idx_v.at[pl.ds(c * _CR + b * _CP, _CP)], sem)
            for c in cs for b in range(_B)]

    head = stage_idx(range(2), sm)
    tail = stage_idx(range(2, _NCHUNK), si)
    for cp in head:
        cp.wait()

    def issue(c, buf):
        return [
            pltpu.async_copy(
                pos_hbm.at[pl.ds(p0 + c * _CP, _CP)], poss[buf], sps[buf]),
            pltpu.async_copy(
                tok_hbm.at[idx_v.at[pl.ds(c * _CR, _CR)]],
                toks[buf], sgs[buf]),
        ]

    issue(0, 0)
    issue(1, 1)

    for cp in tail:
        cp.wait()

    for b in range(_B):
        pltpu.sync_copy(mask_hbm.at[b].at[pl.ds(p0, _PPW)],
                        m_v.at[pl.ds(b * _PPW, _PPW)])

    def mbody(i, _):
        s = pl.ds(i * 16, 16)
        m_v[s] = (1.0 - m_v[s]) * -10000.0
        return _

    lax.fori_loop(0, (_B * _PPW) // 16, mbody, None)
    mask_cps = [pltpu.async_copy(
        m_v.at[pl.ds(b * _PPW, _PPW)],
        mout_hbm.at[b, 0, 0].at[pl.ds(p0, _PPW)], sm)
        for b in range(_B)]

    def drain_outs(c, buf):
        for b in range(_B):
            pltpu.make_async_copy(
                toks[buf].at[pl.ds(b * _CP, _CP)],
                out_hbm.at[b].at[pl.ds(p0 + c * _CP, _CP)],
                sos[buf]).wait()

    def round_body(g, _):
        for phase in range(_NBUF):
            c = 3 * g + phase
            nb = (phase + 2) % _NBUF

            @pl.when(c + 2 < _NCHUNK)
            def _issue():
                @pl.when(c >= 1)
                def _drain():
                    drain_outs(c - 1, nb)
                issue(c + 2, nb)

            @pl.when(c < _NCHUNK)
            def _work():
                pltpu.make_async_copy(
                    pos_hbm.at[pl.ds(p0 + c * _CP, _CP)],
                    poss[phase], sps[phase]).wait()
                pltpu.make_async_copy(
                    tok_hbm.at[idx_v.at[pl.ds(c * _CR, _CR)]],
                    toks[phase], sgs[phase]).wait()

                tok = toks[phase]
                pos = poss[phase]

                @plsc.parallel_loop(0, _CP, 1)
                def abody(r):
                    for j in range(_HV):
                        s = pl.ds(j * 16, 16)
                        v = pos[r, s]
                        for b in range(_B):
                            plsc.addupdate(tok.at[b * _CP + r, s], v)

                for b in range(_B):
                    pltpu.async_copy(
                        tok.at[pl.ds(b * _CP, _CP)],
                        out_hbm.at[b].at[pl.ds(p0 + c * _CP, _CP)],
                        sos[phase])
        return _

    lax.fori_loop(0, (_NCHUNK + _NBUF) // _NBUF, round_body, None)

    drain_outs(_NCHUNK - 3, (_NCHUNK - 3) % _NBUF)
    drain_outs(_NCHUNK - 2, (_NCHUNK - 2) % _NBUF)
    drain_outs(_NCHUNK - 1, (_NCHUNK - 1) % _NBUF)
    for cp in mask_cps:
        cp.wait()


@jax.jit
def _run(input_ids, attention_mask, token_table, pos_table):
    mesh = plsc.VectorSubcoreMesh(core_axis_name="c", subcore_axis_name="s")
    return pl.kernel(
        _body,
        out_type=(
            jax.ShapeDtypeStruct((_B, _S, _H), jnp.float32),
            jax.ShapeDtypeStruct((_B, 1, 1, _S), jnp.float32),
        ),
        mesh=mesh,
        scratch_types=[
            pltpu.VMEM((_B * _PPW,), jnp.int32),
            pltpu.VMEM((_CR, _H), jnp.float32),
            pltpu.VMEM((_CR, _H), jnp.float32),
            pltpu.VMEM((_CR, _H), jnp.float32),
            pltpu.VMEM((_CP, _H), jnp.float32),
            pltpu.VMEM((_CP, _H), jnp.float32),
            pltpu.VMEM((_CP, _H), jnp.float32),
            pltpu.VMEM((_B * _PPW,), jnp.float32),
        ] + [pltpu.SemaphoreType.DMA] * 11,
    )(input_ids, attention_mask, token_table, pos_table)


def kernel(input_ids, attention_mask, token_table, pos_table):
    return _run(input_ids.astype(jnp.int32),
                attention_mask.astype(jnp.float32),
                token_table, pos_table)


# speedup vs baseline: 1.0081x; 1.0081x over previous
import functools

import jax
import jax.numpy as jnp
from jax import lax
from jax.experimental import pallas as pl
from jax.experimental.pallas import tpu as pltpu
from jax.experimental.pallas import tpu_sc as plsc

_INFO = plsc.get_sparse_core_info()
_NC, _NS = _INFO.num_cores, _INFO.num_subcores
_NW = _NC * _NS

_B, _S, _H = 4, 2048, 1024
_PPW = _S // _NW
_CP = 8
_NCHUNK = _PPW // _CP
_CR = _CP * _B
_HV = _H // 16
_NBUF = 3


def _body(ids_hbm, mask_hbm, tok_hbm, pos_hbm, out_hbm, mout_hbm,
          idx_v, tok0, tok1, tok2, pos0, pos1, pos2, m_v,
          sg0, sg1, sg2, sp0, sp1, sp2, so0, so1, so2, sm, si):
    wid = lax.axis_index("s") * _NC + lax.axis_index("c")
    p0 = wid * _PPW

    toks = (tok0, tok1, tok2)
    poss = (pos0, pos1, pos2)
    sgs = (sg0, sg1, sg2)
    sps = (sp0, sp1, sp2)
    sos = (so0, so1, so2)

    def stage_idx(cs, sem):
        return [pltpu.async_copy(
            ids_hbm.at[b].at[pl.ds(p0 + c * _CP, _CP)],
            idx_v.at[pl.ds(c * _CR + b * _CP, _CP)], sem)
            for c in cs for b in range(_B)]

    head = stage_idx(range(2), sm)
    tail = stage_idx(range(2, _NCHUNK), si)
    for cp in head:
        cp.wait()

    def issue(c, buf):
        return [
            pltpu.async_copy(
                pos_hbm.at[pl.ds(p0 + c * _CP, _CP)], poss[buf], sps[buf]),
            pltpu.async_copy(
                tok_hbm.at[idx_v.at[pl.ds(c * _CR, _CR // 2)]],
                toks[buf].at[pl.ds(0, _CR // 2)], sgs[buf]),
            pltpu.async_copy(
                tok_hbm.at[idx_v.at[pl.ds(c * _CR + _CR // 2, _CR // 2)]],
                toks[buf].at[pl.ds(_CR // 2, _CR // 2)], sgs[buf]),
        ]

    issue(0, 0)
    issue(1, 1)

    for cp in tail:
        cp.wait()

    for b in range(_B):
        pltpu.sync_copy(mask_hbm.at[b].at[pl.ds(p0, _PPW)],
                        m_v.at[pl.ds(b * _PPW, _PPW)])

    def mbody(i, _):
        s = pl.ds(i * 16, 16)
        m_v[s] = (1.0 - m_v[s]) * -10000.0
        return _

    lax.fori_loop(0, (_B * _PPW) // 16, mbody, None)
    mask_cps = [pltpu.async_copy(
        m_v.at[pl.ds(b * _PPW, _PPW)],
        mout_hbm.at[b, 0, 0].at[pl.ds(p0, _PPW)], sm)
        for b in range(_B)]

    def drain_outs(c, buf):
        for b in range(_B):
            pltpu.make_async_copy(
                toks[buf].at[pl.ds(b * _CP, _CP)],
                out_hbm.at[b].at[pl.ds(p0 + c * _CP, _CP)],
                sos[buf]).wait()

    def round_body(g, _):
        for phase in range(_NBUF):
            c = 3 * g + phase
            nb = (phase + 2) % _NBUF

            @pl.when(c + 2 < _NCHUNK)
            def _issue():
                @pl.when(c >= 1)
                def _drain():
                    drain_outs(c - 1, nb)
                issue(c + 2, nb)

            @pl.when(c < _NCHUNK)
            def _work():
                pltpu.make_async_copy(
                    pos_hbm.at[pl.ds(p0 + c * _CP, _CP)],
                    poss[phase], sps[phase]).wait()
                pltpu.make_async_copy(
                    tok_hbm.at[idx_v.at[pl.ds(c * _CR, _CR)]],
                    toks[phase], sgs[phase]).wait()

                tok = toks[phase]
                pos = poss[phase]

                def abody(r, _):
                    for j in range(_HV):
                        s = pl.ds(j * 16, 16)
                        v = pos[r, s]
                        for b in range(_B):
                            plsc.addupdate(tok.at[b * _CP + r, s], v)
                    return _

                lax.fori_loop(0, _CP, abody, None)

                for b in range(_B):
                    pltpu.async_copy(
                        tok.at[pl.ds(b * _CP, _CP)],
                        out_hbm.at[b].at[pl.ds(p0 + c * _CP, _CP)],
                        sos[phase])
        return _

    lax.fori_loop(0, (_NCHUNK + _NBUF) // _NBUF, round_body, None)

    drain_outs(_NCHUNK - 3, (_NCHUNK - 3) % _NBUF)
    drain_outs(_NCHUNK - 2, (_NCHUNK - 2) % _NBUF)
    drain_outs(_NCHUNK - 1, (_NCHUNK - 1) % _NBUF)
    for cp in mask_cps:
        cp.wait()


@jax.jit
def _run(input_ids, attention_mask, token_table, pos_table):
    mesh = plsc.VectorSubcoreMesh(core_axis_name="c", subcore_axis_name="s")
    return pl.kernel(
        _body,
        out_type=(
            jax.ShapeDtypeStruct((_B, _S, _H), jnp.float32),
            jax.ShapeDtypeStruct((_B, 1, 1, _S), jnp.float32),
        ),
        mesh=mesh,
        scratch_types=[
            pltpu.VMEM((_B * _PPW,), jnp.int32),
            pltpu.VMEM((_CR, _H), jnp.float32),
            pltpu.VMEM((_CR, _H), jnp.float32),
            pltpu.VMEM((_CR, _H), jnp.float32),
            pltpu.VMEM((_CP, _H), jnp.float32),
            pltpu.VMEM((_CP, _H), jnp.float32),
            pltpu.VMEM((_CP, _H), jnp.float32),
            pltpu.VMEM((_B * _PPW,), jnp.float32),
        ] + [pltpu.SemaphoreType.DMA] * 11,
    )(input_ids, attention_mask, token_table, pos_table)


def kernel(input_ids, attention_mask, token_table, pos_table):
    return _run(input_ids.astype(jnp.int32),
                attention_mask.astype(jnp.float32),
                token_table, pos_table)
